# baseline (speedup 1.0000x reference)
import functools

import jax
import jax.numpy as jnp
from jax import lax
from jax.experimental import pallas as pl
from jax.experimental.pallas import tpu as pltpu
from jax.experimental.pallas import tpu_sc as plsc

N_NODES = 10000
N_PAD = 10240
E_EDGES = 320000
E_PAD = 327680
CHUNKS = E_PAD // 128
CNT_ROWS = CHUNKS // 32
AGG_ROWS = CHUNKS // 16
NROWS_TILE = N_PAD // 16

@functools.cache
def _mesh():
    return plsc.VectorSubcoreMesh(
        core_axis_name="c", subcore_axis_name="s", num_cores=2, num_subcores=16
    )


@functools.cache
def _make_counts():
    @functools.partial(
        pl.kernel,
        out_type=jax.ShapeDtypeStruct((2, N_PAD, 16), jnp.float32),
        mesh=_mesh(),
        scratch_types=[
            pltpu.VMEM_SHARED((N_PAD, 16), jnp.float32),
            pltpu.VMEM((CNT_ROWS, 128), jnp.int32),
            pltpu.VMEM((128, 16), jnp.float32),
            pltpu.VMEM((16, 16), jnp.float32),
            pltpu.VMEM((NROWS_TILE, 16), jnp.float32),
        ],
    )
    def _counts_kernel(col_hbm, out_hbm, cnt_sh, colbuf, ones_buf, zbuf, obuf):
        c = lax.axis_index("c")
        s = lax.axis_index("s")
        w = c * 16 + s

        @pl.loop(0, 128)
        def _fill_ones(i):
            ones_buf[i, :] = jnp.ones((16,), jnp.float32)

        @pl.loop(0, 16)
        def _fill_zero(i):
            zbuf[i, :] = jnp.zeros((16,), jnp.float32)

        @pl.loop(0, NROWS_TILE // 16)
        def _zero_acc(k):
            pltpu.sync_copy(zbuf, cnt_sh.at[pl.ds(s * NROWS_TILE + k * 16, 16)])

        pltpu.sync_copy(col_hbm.at[pl.ds(w * CNT_ROWS, CNT_ROWS)], colbuf)
        plsc.subcore_barrier()

        @pl.loop(0, CNT_ROWS)
        def _accumulate(j):
            pltpu.sync_copy(ones_buf, cnt_sh.at[colbuf.at[j]], add=True)

        plsc.subcore_barrier()
        pltpu.sync_copy(cnt_sh.at[pl.ds(s * NROWS_TILE, NROWS_TILE)], obuf)
        pltpu.sync_copy(obuf, out_hbm.at[c, pl.ds(s * NROWS_TILE, NROWS_TILE)])

    return _counts_kernel


@functools.cache
def _make_agg(dh):
    nbuf = 4 if dh == 64 else 8
    ngroups = AGG_ROWS // nbuf

    @functools.partial(
        pl.kernel,
        out_type=jax.ShapeDtypeStruct((2, N_PAD, dh), jnp.float32),
        mesh=_mesh(),
        compiler_params=pltpu.CompilerParams(use_tc_tiling_on_sc=False),
        scratch_types=[
            pltpu.VMEM_SHARED((N_PAD, dh), jnp.float32),
            pltpu.VMEM_SHARED((N_PAD, dh), jnp.float32),
            pltpu.VMEM((2, 8, 128), jnp.int32),
            pltpu.VMEM((2, 8, 128), jnp.int32),
            pltpu.VMEM((nbuf, 128, dh), jnp.float32),
            pltpu.VMEM((16, dh), jnp.float32),
            pltpu.SemaphoreType.DMA((nbuf,)),
            pltpu.SemaphoreType.DMA((nbuf,)),
            pltpu.SemaphoreType.DMA((2,)),
            pltpu.SemaphoreType.DMA((2,)),
        ],
    )
    def _agg(g_hbm, row_hbm, col_hbm, out_hbm, acc_sh, gtab, ibufr, ibufc,
             gbufs, zbuf, gsem, ssem, isemr, isemc):
        c = lax.axis_index("c")
        s = lax.axis_index("s")

        @pl.loop(0, 16)
        def _fill_zero(i):
            for j2 in range(dh // 16):
                zbuf[i, pl.ds(j2 * 16, 16)] = jnp.zeros((16,), jnp.float32)

        @pl.loop(0, NROWS_TILE // 16)
        def _zero_acc(k):
            pltpu.sync_copy(zbuf, acc_sh.at[pl.ds(s * NROWS_TILE + k * 16, 16)])

        pltpu.sync_copy(g_hbm.at[c, pl.ds(s * NROWS_TILE, NROWS_TILE)],
                        gtab.at[pl.ds(s * NROWS_TILE, NROWS_TILE)])

        def _idx_copies(g, slot):
            off = s * AGG_ROWS + g * nbuf
            return (
                pltpu.make_async_copy(row_hbm.at[pl.ds(off, nbuf)],
                                      ibufr.at[slot, pl.ds(0, nbuf)],
                                      isemr.at[slot]),
                pltpu.make_async_copy(col_hbm.at[pl.ds(off, nbuf)],
                                      ibufc.at[slot, pl.ds(0, nbuf)],
                                      isemc.at[slot]),
            )

        for d in _idx_copies(0, 0):
            d.start()
        plsc.subcore_barrier()

        def _group(g, p, prefetch):
            for d in _idx_copies(g, p):
                d.wait()
            gd = [
                pltpu.async_copy(gtab.at[ibufr.at[p, b]],
                                 gbufs.at[b], gsem.at[b])
                for b in range(nbuf)
            ]
            if prefetch:
                for d in _idx_copies(g + 1, 1 - p):
                    d.start()
            sd = []
            for b in range(nbuf):
                gd[b].wait()
                sd.append(
                    pltpu.async_copy(gbufs.at[b], acc_sh.at[ibufc.at[p, b]],
                                     ssem.at[b], add=True))
            for b in range(nbuf):
                sd[b].wait()

        @pl.loop(0, ngroups // 2 - 1)
        def _pair(t):
            _group(2 * t, 0, True)
            _group(2 * t + 1, 1, True)

        _group(ngroups - 2, 0, True)
        _group(ngroups - 1, 1, False)

        plsc.subcore_barrier()

        @pl.loop(0, NROWS_TILE // 128)
        def _out(k):
            pltpu.sync_copy(acc_sh.at[pl.ds(s * NROWS_TILE + k * 128, 128)],
                            gbufs.at[0])
            pltpu.sync_copy(
                gbufs.at[0], out_hbm.at[c, pl.ds(s * NROWS_TILE + k * 128, 128)])

    return _agg


_BM = 1280
_GRID = N_PAD // _BM


def _b1a_body(x_ref, w1_ref, h_ref):
    h_ref[...] = jnp.dot(x_ref[...], w1_ref[...],
                         preferred_element_type=jnp.float32)


def _b1b_body(h_ref, cnt_ref, g1_ref):
    deg = cnt_ref[0, :, 0:1] + cnt_ref[1, :, 0:1] + 1.0
    g = h_ref[...] * lax.rsqrt(deg)
    g1_ref[...] = jnp.stack([g[:, :64], g[:, 64:]], axis=0)


def _b2_body(acc_ref, g1_ref, cnt_ref, b1_ref, w2_ref, g2_ref):
    deg = cnt_ref[0, :, 0:1] + cnt_ref[1, :, 0:1] + 1.0
    dis = lax.rsqrt(deg)
    st = acc_ref[...] + g1_ref[...]
    h1 = jnp.concatenate([st[0], st[1]], axis=1)
    o1 = jnp.maximum(h1 * dis + b1_ref[...], 0.0)
    h2 = jnp.dot(o1, w2_ref[...], preferred_element_type=jnp.float32)
    g2 = h2 * dis
    g2_ref[...] = jnp.stack([g2[:, :32], g2[:, 32:]], axis=0)


def _b3_body(acc_ref, g2_ref, cnt_ref, b2_ref, out_ref):
    deg = cnt_ref[0, :, 0:1] + cnt_ref[1, :, 0:1] + 1.0
    st = acc_ref[...] + g2_ref[...]
    o = jnp.concatenate([st[0], st[1]], axis=1)
    out_ref[...] = o * lax.rsqrt(deg) + b2_ref[...]


def _row_spec(d):
    return pl.BlockSpec((_BM, d), lambda i: (i, 0))


def _split_spec(d):
    return pl.BlockSpec((2, _BM, d), lambda i: (0, i, 0))


def _full_spec(shape):
    nd = len(shape)
    return pl.BlockSpec(shape, lambda i: (0,) * nd)


_b1a = pl.pallas_call(
    _b1a_body,
    grid=(_GRID,),
    in_specs=[_row_spec(128), _full_spec((128, 128))],
    out_specs=_row_spec(128),
    out_shape=jax.ShapeDtypeStruct((N_PAD, 128), jnp.float32),
)

_b1b = pl.pallas_call(
    _b1b_body,
    grid=(_GRID,),
    in_specs=[_row_spec(128), _split_spec(16)],
    out_specs=_split_spec(64),
    out_shape=jax.ShapeDtypeStruct((2, N_PAD, 64), jnp.float32),
)

_b2 = pl.pallas_call(
    _b2_body,
    grid=(_GRID,),
    in_specs=[_split_spec(64), _split_spec(64), _split_spec(16),
              _full_spec((1, 128)), _full_spec((128, 64))],
    out_specs=_split_spec(32),
    out_shape=jax.ShapeDtypeStruct((2, N_PAD, 32), jnp.float32),
)

_b3 = pl.pallas_call(
    _b3_body,
    grid=(_GRID,),
    in_specs=[_split_spec(32), _split_spec(32), _split_spec(16),
              _full_spec((1, 64))],
    out_specs=_row_spec(64),
    out_shape=jax.ShapeDtypeStruct((N_PAD, 64), jnp.float32),
)


def kernel(x, edge_index, W1, b1, W2, b2):
    row = edge_index[0].astype(jnp.int32)
    col = edge_index[1].astype(jnp.int32)
    pad = E_PAD - E_EDGES
    row2 = jnp.concatenate([row, jnp.zeros((pad,), jnp.int32)]).reshape(CHUNKS, 128)
    col2 = jnp.concatenate([col, jnp.full((pad,), N_NODES, jnp.int32)]).reshape(CHUNKS, 128)
    xp = jnp.pad(x, ((0, N_PAD - N_NODES), (0, 0)))

    cnt = _make_counts()(col2)
    h1 = _b1a(xp, W1)
    g1 = _b1b(h1, cnt)
    acc1 = _make_agg(64)(g1, row2, col2)
    g2 = _b2(acc1, g1, cnt, b1.reshape(1, 128), W2)
    acc2 = _make_agg(32)(g2, row2, col2)
    out = _b3(acc2, g2, cnt, b2.reshape(1, 64))
    return out[:N_NODES]


# speedup vs baseline: 1.0373x; 1.0373x over previous
import functools

import jax
import jax.numpy as jnp
from jax import lax
from jax.experimental import pallas as pl
from jax.experimental.pallas import tpu as pltpu
from jax.experimental.pallas import tpu_sc as plsc

N_NODES = 10000
N_PAD = 10240
E_EDGES = 320000
E_PAD = 327680
CHUNKS = E_PAD // 128
CNT_ROWS = CHUNKS // 32
AGG_ROWS = CHUNKS // 16
NROWS_TILE = N_PAD // 16

@functools.cache
def _mesh():
    return plsc.VectorSubcoreMesh(
        core_axis_name="c", subcore_axis_name="s", num_cores=2, num_subcores=16
    )


@functools.cache
def _make_counts():
    @functools.partial(
        pl.kernel,
        out_type=jax.ShapeDtypeStruct((2, N_PAD, 16), jnp.float32),
        mesh=_mesh(),
        scratch_types=[
            pltpu.VMEM_SHARED((N_PAD, 16), jnp.float32),
            pltpu.VMEM((CNT_ROWS, 128), jnp.int32),
            pltpu.VMEM((128, 16), jnp.float32),
            pltpu.VMEM((16, 16), jnp.float32),
            pltpu.VMEM((NROWS_TILE, 16), jnp.float32),
            pltpu.SemaphoreType.DMA((4,)),
        ],
    )
    def _counts_kernel(col_hbm, out_hbm, cnt_sh, colbuf, ones_buf, zbuf, obuf,
                       csem):
        c = lax.axis_index("c")
        s = lax.axis_index("s")
        w = c * 16 + s

        @pl.loop(0, 128)
        def _fill_ones(i):
            ones_buf[i, :] = jnp.ones((16,), jnp.float32)

        @pl.loop(0, 16)
        def _fill_zero(i):
            zbuf[i, :] = jnp.zeros((16,), jnp.float32)

        @pl.loop(0, NROWS_TILE // 16)
        def _zero_acc(k):
            pltpu.sync_copy(zbuf, cnt_sh.at[pl.ds(s * NROWS_TILE + k * 16, 16)])

        pltpu.sync_copy(col_hbm.at[pl.ds(w * CNT_ROWS, CNT_ROWS)], colbuf)
        plsc.subcore_barrier()

        @pl.loop(0, CNT_ROWS)
        def _accumulate(j):
            pltpu.sync_copy(ones_buf, cnt_sh.at[colbuf.at[j]], add=True)

        plsc.subcore_barrier()
        pltpu.sync_copy(cnt_sh.at[pl.ds(s * NROWS_TILE, NROWS_TILE)], obuf)
        pltpu.sync_copy(obuf, out_hbm.at[c, pl.ds(s * NROWS_TILE, NROWS_TILE)])

    return _counts_kernel


@functools.cache
def _make_agg(dh):
    nbuf = 4 if dh == 64 else 8
    ngroups = AGG_ROWS // nbuf

    @functools.partial(
        pl.kernel,
        out_type=jax.ShapeDtypeStruct((2, N_PAD, dh), jnp.float32),
        mesh=_mesh(),
        compiler_params=pltpu.CompilerParams(use_tc_tiling_on_sc=False),
        scratch_types=[
            pltpu.VMEM_SHARED((N_PAD, dh), jnp.float32),
            pltpu.VMEM_SHARED((N_PAD, dh), jnp.float32),
            pltpu.VMEM((2, 8, 128), jnp.int32),
            pltpu.VMEM((2, 8, 128), jnp.int32),
            pltpu.VMEM((nbuf, 128, dh), jnp.float32),
            pltpu.VMEM((16, dh), jnp.float32),
            pltpu.SemaphoreType.DMA((nbuf,)),
            pltpu.SemaphoreType.DMA((nbuf,)),
            pltpu.SemaphoreType.DMA((2,)),
            pltpu.SemaphoreType.DMA((2,)),
        ],
    )
    def _agg(g_hbm, row_hbm, col_hbm, out_hbm, acc_sh, gtab, ibufr, ibufc,
             gbufs, zbuf, gsem, ssem, isemr, isemc):
        c = lax.axis_index("c")
        s = lax.axis_index("s")

        stg = pltpu.async_copy(g_hbm.at[c, pl.ds(s * NROWS_TILE, NROWS_TILE)],
                               gtab.at[pl.ds(s * NROWS_TILE, NROWS_TILE)],
                               gsem.at[0])

        @pl.loop(0, 16)
        def _fill_zero(i):
            for j2 in range(dh // 16):
                zbuf[i, pl.ds(j2 * 16, 16)] = jnp.zeros((16,), jnp.float32)

        @pl.loop(0, NROWS_TILE // 16)
        def _zero_acc(k):
            pltpu.sync_copy(zbuf, acc_sh.at[pl.ds(s * NROWS_TILE + k * 16, 16)])

        stg.wait()

        def _idx_copies(g, slot):
            off = s * AGG_ROWS + g * nbuf
            return (
                pltpu.make_async_copy(row_hbm.at[pl.ds(off, nbuf)],
                                      ibufr.at[slot, pl.ds(0, nbuf)],
                                      isemr.at[slot]),
                pltpu.make_async_copy(col_hbm.at[pl.ds(off, nbuf)],
                                      ibufc.at[slot, pl.ds(0, nbuf)],
                                      isemc.at[slot]),
            )

        for d in _idx_copies(0, 0):
            d.start()
        plsc.subcore_barrier()

        def _group(g, p, prefetch):
            for d in _idx_copies(g, p):
                d.wait()
            gd = [
                pltpu.async_copy(gtab.at[ibufr.at[p, b]],
                                 gbufs.at[b], gsem.at[b])
                for b in range(nbuf)
            ]
            if prefetch:
                for d in _idx_copies(g + 1, 1 - p):
                    d.start()
            sd = []
            for b in range(nbuf):
                gd[b].wait()
                sd.append(
                    pltpu.async_copy(gbufs.at[b], acc_sh.at[ibufc.at[p, b]],
                                     ssem.at[b], add=True))
            for b in range(nbuf):
                sd[b].wait()

        @pl.loop(0, ngroups // 2 - 1)
        def _pair(t):
            _group(2 * t, 0, True)
            _group(2 * t + 1, 1, True)

        _group(ngroups - 2, 0, True)
        _group(ngroups - 1, 1, False)

        plsc.subcore_barrier()

        nseg = NROWS_TILE // 128
        wrs = [None] * nseg
        for k in range(nseg):
            buf = k % 2
            if k >= 2:
                wrs[k - 2].wait()
            pltpu.async_copy(acc_sh.at[pl.ds(s * NROWS_TILE + k * 128, 128)],
                             gbufs.at[buf], gsem.at[buf]).wait()
            wrs[k] = pltpu.async_copy(
                gbufs.at[buf], out_hbm.at[c, pl.ds(s * NROWS_TILE + k * 128, 128)],
                ssem.at[buf])
        wrs[nseg - 2].wait()
        wrs[nseg - 1].wait()

    return _agg


_BM = 1280
_GRID = N_PAD // _BM


def _b1_body(x_ref, w1_ref, cnt_ref, g1_ref):
    h = jnp.dot(x_ref[...], w1_ref[...], preferred_element_type=jnp.float32)
    deg = cnt_ref[0, :, 0:1] + cnt_ref[1, :, 0:1] + 1.0
    g = h * lax.rsqrt(deg)
    g1_ref[...] = jnp.stack([g[:, :64], g[:, 64:]], axis=0)


def _b2_body(acc_ref, g1_ref, cnt_ref, b1_ref, w2_ref, g2_ref):
    deg = cnt_ref[0, :, 0:1] + cnt_ref[1, :, 0:1] + 1.0
    dis = lax.rsqrt(deg)
    st = acc_ref[...] + g1_ref[...]
    h1 = jnp.concatenate([st[0], st[1]], axis=1)
    o1 = jnp.maximum(h1 * dis + b1_ref[...], 0.0)
    h2 = jnp.dot(o1, w2_ref[...], preferred_element_type=jnp.float32)
    g2 = h2 * dis
    g2_ref[...] = jnp.stack([g2[:, :32], g2[:, 32:]], axis=0)


def _b3_body(acc_ref, g2_ref, cnt_ref, b2_ref, out_ref):
    deg = cnt_ref[0, :, 0:1] + cnt_ref[1, :, 0:1] + 1.0
    st = acc_ref[...] + g2_ref[...]
    o = jnp.concatenate([st[0], st[1]], axis=1)
    out_ref[...] = o * lax.rsqrt(deg) + b2_ref[...]


def _row_spec(d):
    return pl.BlockSpec((_BM, d), lambda i: (i, 0))


def _split_spec(d):
    return pl.BlockSpec((2, _BM, d), lambda i: (0, i, 0))


def _full_spec(shape):
    nd = len(shape)
    return pl.BlockSpec(shape, lambda i: (0,) * nd)


_b1 = pl.pallas_call(
    _b1_body,
    grid=(_GRID,),
    in_specs=[_row_spec(128), _full_spec((128, 128)), _split_spec(16)],
    out_specs=_split_spec(64),
    out_shape=jax.ShapeDtypeStruct((2, N_PAD, 64), jnp.float32),
)

_b2 = pl.pallas_call(
    _b2_body,
    grid=(_GRID,),
    in_specs=[_split_spec(64), _split_spec(64), _split_spec(16),
              _full_spec((1, 128)), _full_spec((128, 64))],
    out_specs=_split_spec(32),
    out_shape=jax.ShapeDtypeStruct((2, N_PAD, 32), jnp.float32),
)

_b3 = pl.pallas_call(
    _b3_body,
    grid=(_GRID,),
    in_specs=[_split_spec(32), _split_spec(32), _split_spec(16),
              _full_spec((1, 64))],
    out_specs=_row_spec(64),
    out_shape=jax.ShapeDtypeStruct((N_PAD, 64), jnp.float32),
)


def kernel(x, edge_index, W1, b1, W2, b2):
    row = edge_index[0].astype(jnp.int32)
    col = edge_index[1].astype(jnp.int32)
    pad = E_PAD - E_EDGES
    row2 = jnp.concatenate([row, jnp.zeros((pad,), jnp.int32)]).reshape(CHUNKS, 128)
    col2 = jnp.concatenate([col, jnp.full((pad,), N_NODES, jnp.int32)]).reshape(CHUNKS, 128)
    xp = jnp.pad(x, ((0, N_PAD - N_NODES), (0, 0)))

    cnt = _make_counts()(col2)
    g1 = _b1(xp, W1, cnt)
    acc1 = _make_agg(64)(g1, row2, col2)
    g2 = _b2(acc1, g1, cnt, b1.reshape(1, 128), W2)
    acc2 = _make_agg(32)(g2, row2, col2)
    out = _b3(acc2, g2, cnt, b2.reshape(1, 64))
    return out[:N_NODES]
